# baseline (device time: 40513 ns/iter reference)
import jax
import jax.numpy as jnp
from jax import lax
from jax.experimental import pallas as pl
from jax.experimental.pallas import tpu as pltpu

N_DEV = 8
B = 2
S_PER = 128
S_GLOB = N_DEV * S_PER
HQ = 4
DH = 64
LOCAL_WINDOW = 128
GLOBAL_PREFIX = 32
R_HOPS = 4
L_HOPS = 3


def kernel(x, Wq, K_ext, V_ext, Wo):
    d_model = x.shape[-1]

    def body(x_ref, wq_ref, k_ref, v_ref, wo_ref, out_ref,
             kvh_ref, r_send, r_recv, l_send, l_recv):
        my = lax.axis_index("i")
        left = lax.rem(my - 1 + N_DEV, N_DEV)
        right = lax.rem(my + 1, N_DEV)

        barrier_sem = pltpu.get_barrier_semaphore()
        for nbr in (left, right):
            pl.semaphore_signal(
                barrier_sem, inc=1,
                device_id=(nbr,), device_id_type=pl.DeviceIdType.MESH,
            )
        pl.semaphore_wait(barrier_sem, 2)

        for j in range(HQ):
            kvh_ref[j, :, pl.ds(my * S_PER, S_PER), :] = (
                k_ref[:, :, j, :].astype(jnp.bfloat16))
            kvh_ref[HQ + j, :, pl.ds(my * S_PER, S_PER), :] = (
                v_ref[:, :, j, :].astype(jnp.bfloat16))

        def block(origin):
            return kvh_ref.at[:, :, pl.ds(origin * S_PER, S_PER), :]

        def make_rdma(origin, sems_s, sems_r, h, target):
            return pltpu.make_async_remote_copy(
                src_ref=block(origin), dst_ref=block(origin),
                send_sem=sems_s.at[h], recv_sem=sems_r.at[h],
                device_id=(target,), device_id_type=pl.DeviceIdType.MESH,
            )

        def org(k):
            return lax.rem(my + k + N_DEV, N_DEV)

        r_rdma = [make_rdma(org(-h), r_send, r_recv, h, right)
                  for h in range(R_HOPS)]
        l_rdma = [make_rdma(org(+h), l_send, l_recv, h, left)
                  for h in range(L_HOPS)]
        r_rdma[0].start()
        l_rdma[0].start()

        xb = x_ref[:].astype(jnp.bfloat16)
        wq = wq_ref[:].astype(jnp.bfloat16)
        q = lax.dot_general(
            xb, wq, (((2,), (0,)), ((), ())),
            preferred_element_type=jnp.float32,
        )
        qh = [q[:, :, h * DH:(h + 1) * DH].astype(jnp.bfloat16)
              for h in range(HQ)]

        qi_loc = lax.broadcasted_iota(jnp.int32, (S_PER, S_PER), 0)
        ki_loc = lax.broadcasted_iota(jnp.int32, (S_PER, S_PER), 1)
        qi_glob = qi_loc + my * S_PER

        num = [jnp.zeros((B, S_PER, DH), jnp.float32) for _ in range(HQ)]
        den = [jnp.zeros((B, S_PER, 1), jnp.float32) for _ in range(HQ)]
        num32 = [jnp.zeros((B, 32, DH), jnp.float32) for _ in range(HQ)]
        den32 = [jnp.zeros((B, 32, 1), jnp.float32) for _ in range(HQ)]

        def accumulate(origin):
            ko = origin * S_PER
            ki_glob = ki_loc + ko
            mask = ((jnp.abs(qi_glob - ki_glob) <= LOCAL_WINDOW)
                    | (ki_glob < GLOBAL_PREFIX) | (qi_glob < GLOBAL_PREFIX))
            maskf = mask.astype(jnp.float32)[None, :, :]
            for h in range(HQ):
                kb = kvh_ref[h, :, pl.ds(ko, S_PER), :]
                vb = kvh_ref[HQ + h, :, pl.ds(ko, S_PER), :]
                s = lax.dot_general(
                    qh[h], kb, (((2,), (2,)), ((0,), (0,))),
                    preferred_element_type=jnp.float32,
                ) * 0.125
                e = jnp.exp(s) * maskf
                num[h] = num[h] + lax.dot_general(
                    e.astype(jnp.bfloat16), vb, (((2,), (1,)), ((0,), (0,))),
                    preferred_element_type=jnp.float32,
                )
                den[h] = den[h] + jnp.sum(e, axis=-1, keepdims=True)

        def accumulate_far(origin):
            ko = origin * S_PER
            ind_a = jnp.where(my == 0, 1.0, 0.0).astype(jnp.float32)
            ind_b = jnp.where(origin == 0, 1.0, 0.0).astype(jnp.float32)
            for h in range(HQ):
                kb = kvh_ref[h, :, pl.ds(ko, 32), :]
                vb = kvh_ref[HQ + h, :, pl.ds(ko, 32), :]
                s = lax.dot_general(
                    qh[h], kb, (((2,), (2,)), ((0,), (0,))),
                    preferred_element_type=jnp.float32,
                ) * 0.125
                e = jnp.exp(s) * ind_b
                num[h] = num[h] + lax.dot_general(
                    e.astype(jnp.bfloat16), vb, (((2,), (1,)), ((0,), (0,))),
                    preferred_element_type=jnp.float32,
                )
                den[h] = den[h] + jnp.sum(e, axis=-1, keepdims=True)
                kf = kvh_ref[h, :, pl.ds(ko, S_PER), :]
                vf = kvh_ref[HQ + h, :, pl.ds(ko, S_PER), :]
                s32 = lax.dot_general(
                    qh[h][:, :32, :], kf, (((2,), (2,)), ((0,), (0,))),
                    preferred_element_type=jnp.float32,
                ) * 0.125
                e32 = jnp.exp(s32) * ind_a
                num32[h] = num32[h] + lax.dot_general(
                    e32.astype(jnp.bfloat16), vf, (((2,), (1,)), ((0,), (0,))),
                    preferred_element_type=jnp.float32,
                )
                den32[h] = den32[h] + jnp.sum(e32, axis=-1, keepdims=True)

        accumulate(org(0))

        for h in range(R_HOPS):
            r_rdma[h].wait_recv()
            if h + 1 < R_HOPS:
                r_rdma[h + 1].start()
            if h < L_HOPS:
                l_rdma[h].wait_recv()
                if h + 1 < L_HOPS:
                    l_rdma[h + 1].start()
            if h == 0:
                accumulate(org(-1))
                accumulate(org(+1))
            else:
                accumulate_far(org(-1 - h))
                if h < L_HOPS:
                    accumulate_far(org(+1 + h))

        for r in r_rdma:
            r.wait_send()
        for r in l_rdma:
            r.wait_send()

        acc = jnp.zeros((B, S_PER, d_model), dtype=jnp.float32)
        for h in range(HQ):
            n = jnp.concatenate(
                [num[h][:, :32, :] + num32[h], num[h][:, 32:, :]], axis=1)
            d = jnp.concatenate(
                [den[h][:, :32, :] + den32[h], den[h][:, 32:, :]], axis=1)
            ctx = (n / d).astype(jnp.bfloat16)
            woh = wo_ref[h * DH:(h + 1) * DH, :].astype(jnp.bfloat16)
            acc = acc + lax.dot_general(
                ctx, woh, (((2,), (0,)), ((), ())),
                preferred_element_type=jnp.float32,
            )
        out_ref[:] = acc

    out_shape = jax.ShapeDtypeStruct((B, S_PER, d_model), jnp.float32)
    return pl.pallas_call(
        body,
        out_shape=out_shape,
        in_specs=[pl.BlockSpec(memory_space=pltpu.VMEM)] * 5,
        out_specs=pl.BlockSpec(memory_space=pltpu.VMEM),
        scratch_shapes=[
            pltpu.VMEM((2 * HQ, B, S_GLOB, DH), jnp.bfloat16),
            pltpu.SemaphoreType.DMA((R_HOPS,)),
            pltpu.SemaphoreType.DMA((R_HOPS,)),
            pltpu.SemaphoreType.DMA((L_HOPS,)),
            pltpu.SemaphoreType.DMA((L_HOPS,)),
        ],
        compiler_params=pltpu.CompilerParams(collective_id=0),
    )(x, Wq, K_ext, V_ext, Wo)


# device time: 40409 ns/iter; 1.0026x vs baseline; 1.0026x over previous
import jax
import jax.numpy as jnp
from jax import lax
from jax.experimental import pallas as pl
from jax.experimental.pallas import tpu as pltpu

N_DEV = 8
B = 2
S_PER = 128
S_GLOB = N_DEV * S_PER
HQ = 4
DH = 64
G = HQ * B
LOCAL_WINDOW = 128
GLOBAL_PREFIX = 32
R_HOPS = 4
L_HOPS = 3


def kernel(x, Wq, K_ext, V_ext, Wo):
    d_model = x.shape[-1]

    def body(x_ref, wq_ref, k_ref, v_ref, wo_ref, out_ref,
             kvh_ref, r_send, r_recv, l_send, l_recv):
        my = lax.axis_index("i")
        left = lax.rem(my - 1 + N_DEV, N_DEV)
        right = lax.rem(my + 1, N_DEV)

        barrier_sem = pltpu.get_barrier_semaphore()
        for nbr in (left, right):
            pl.semaphore_signal(
                barrier_sem, inc=1,
                device_id=(nbr,), device_id_type=pl.DeviceIdType.MESH,
            )
        pl.semaphore_wait(barrier_sem, 2)

        for j in range(HQ):
            kvh_ref[pl.ds(j * B, B), pl.ds(my * S_PER, S_PER), :] = (
                k_ref[:, :, j, :].astype(jnp.bfloat16))
            kvh_ref[pl.ds(G + j * B, B), pl.ds(my * S_PER, S_PER), :] = (
                v_ref[:, :, j, :].astype(jnp.bfloat16))

        def block(origin):
            return kvh_ref.at[:, pl.ds(origin * S_PER, S_PER), :]

        def make_rdma(origin, sems_s, sems_r, h, target):
            return pltpu.make_async_remote_copy(
                src_ref=block(origin), dst_ref=block(origin),
                send_sem=sems_s.at[h], recv_sem=sems_r.at[h],
                device_id=(target,), device_id_type=pl.DeviceIdType.MESH,
            )

        def org(k):
            return lax.rem(my + k + N_DEV, N_DEV)

        r_rdma = [make_rdma(org(-h), r_send, r_recv, h, right)
                  for h in range(R_HOPS)]
        l_rdma = [make_rdma(org(+h), l_send, l_recv, h, left)
                  for h in range(L_HOPS)]
        r_rdma[0].start()
        l_rdma[0].start()

        xb = x_ref[:].astype(jnp.bfloat16)
        qall = jnp.concatenate([
            lax.dot_general(
                xb, wq_ref[:, h * DH:(h + 1) * DH].astype(jnp.bfloat16),
                (((2,), (0,)), ((), ())),
                preferred_element_type=jnp.float32,
            ).astype(jnp.bfloat16)
            for h in range(HQ)
        ], axis=0)

        qi_loc = lax.broadcasted_iota(jnp.int32, (S_PER, S_PER), 0)
        ki_loc = lax.broadcasted_iota(jnp.int32, (S_PER, S_PER), 1)
        qi_glob = qi_loc + my * S_PER

        num = jnp.zeros((G, S_PER, DH), jnp.float32)
        den = jnp.zeros((G, S_PER, 1), jnp.float32)

        def accumulate_near(origin):
            nonlocal num, den
            ko = origin * S_PER
            ki_glob = ki_loc + ko
            mask = ((jnp.abs(qi_glob - ki_glob) <= LOCAL_WINDOW)
                    | (ki_glob < GLOBAL_PREFIX) | (qi_glob < GLOBAL_PREFIX))
            maskf = mask.astype(jnp.float32)[None, :, :]
            kb = kvh_ref[0:G, pl.ds(ko, S_PER), :]
            vb = kvh_ref[G:2 * G, pl.ds(ko, S_PER), :]
            s = lax.dot_general(
                qall, kb, (((2,), (2,)), ((0,), (0,))),
                preferred_element_type=jnp.float32,
            ) * 0.125
            e = jnp.exp(s) * maskf
            num = num + lax.dot_general(
                e.astype(jnp.bfloat16), vb, (((2,), (1,)), ((0,), (0,))),
                preferred_element_type=jnp.float32,
            )
            den = den + jnp.sum(e, axis=-1, keepdims=True)

        accumulate_near(org(0))

        for h in range(R_HOPS):
            r_rdma[h].wait_recv()
            if h + 1 < R_HOPS:
                r_rdma[h + 1].start()
            if h < L_HOPS:
                l_rdma[h].wait_recv()
                if h + 1 < L_HOPS:
                    l_rdma[h + 1].start()
            if h == 0:
                accumulate_near(org(-1))
                accumulate_near(org(+1))

        for r in r_rdma:
            r.wait_send()
        for r in l_rdma:
            r.wait_send()

        ind_b = jnp.where((my >= 2) & (my <= 6), 1.0, 0.0)
        kb0 = kvh_ref[0:G, 0:GLOBAL_PREFIX, :]
        vb0 = kvh_ref[G:2 * G, 0:GLOBAL_PREFIX, :]
        sb = lax.dot_general(
            qall, kb0, (((2,), (2,)), ((0,), (0,))),
            preferred_element_type=jnp.float32,
        ) * 0.125
        eb = jnp.exp(sb) * ind_b
        num = num + lax.dot_general(
            eb.astype(jnp.bfloat16), vb0, (((2,), (1,)), ((0,), (0,))),
            preferred_element_type=jnp.float32,
        )
        den = den + jnp.sum(eb, axis=-1, keepdims=True)

        ind_a = jnp.where(my == 0, 1.0, 0.0)
        ki_full = lax.broadcasted_iota(jnp.int32, (GLOBAL_PREFIX, S_GLOB), 1)
        o_col = ki_full // S_PER
        d1 = lax.rem(o_col - my + N_DEV, N_DEV)
        far = jnp.minimum(d1, N_DEV - d1) >= 2
        farf = far.astype(jnp.float32)[None, :, :]
        kf = kvh_ref[0:G]
        vf = kvh_ref[G:2 * G]
        sa = lax.dot_general(
            qall[:, :GLOBAL_PREFIX, :], kf, (((2,), (2,)), ((0,), (0,))),
            preferred_element_type=jnp.float32,
        ) * 0.125
        ea = jnp.exp(sa) * (ind_a * farf)
        num32 = lax.dot_general(
            ea.astype(jnp.bfloat16), vf, (((2,), (1,)), ((0,), (0,))),
            preferred_element_type=jnp.float32,
        )
        den32 = jnp.sum(ea, axis=-1, keepdims=True)

        n = jnp.concatenate(
            [num[:, :GLOBAL_PREFIX, :] + num32,
             num[:, GLOBAL_PREFIX:, :]], axis=1)
        d = jnp.concatenate(
            [den[:, :GLOBAL_PREFIX, :] + den32,
             den[:, GLOBAL_PREFIX:, :]], axis=1)
        ctx = (n / d).astype(jnp.bfloat16)
        acc = jnp.zeros((B, S_PER, d_model), dtype=jnp.float32)
        for h in range(HQ):
            woh = wo_ref[h * DH:(h + 1) * DH, :].astype(jnp.bfloat16)
            acc = acc + lax.dot_general(
                ctx[h * B:(h + 1) * B], woh, (((2,), (0,)), ((), ())),
                preferred_element_type=jnp.float32,
            )
        out_ref[:] = acc

    out_shape = jax.ShapeDtypeStruct((B, S_PER, d_model), jnp.float32)
    return pl.pallas_call(
        body,
        out_shape=out_shape,
        in_specs=[pl.BlockSpec(memory_space=pltpu.VMEM)] * 5,
        out_specs=pl.BlockSpec(memory_space=pltpu.VMEM),
        scratch_shapes=[
            pltpu.VMEM((2 * G, S_GLOB, DH), jnp.bfloat16),
            pltpu.SemaphoreType.DMA((R_HOPS,)),
            pltpu.SemaphoreType.DMA((R_HOPS,)),
            pltpu.SemaphoreType.DMA((L_HOPS,)),
            pltpu.SemaphoreType.DMA((L_HOPS,)),
        ],
        compiler_params=pltpu.CompilerParams(collective_id=0),
    )(x, Wq, K_ext, V_ext, Wo)


# device time: 28728 ns/iter; 1.4102x vs baseline; 1.4066x over previous
import jax
import jax.numpy as jnp
from jax import lax
from jax.experimental import pallas as pl
from jax.experimental.pallas import tpu as pltpu

N_DEV = 8
B = 2
S_PER = 128
S_GLOB = N_DEV * S_PER
HQ = 4
DH = 64
G = HQ * B
S_NEAR = 3 * S_PER
LOCAL_WINDOW = 128
GLOBAL_PREFIX = 32
MIDS = (2, 3, 4, 5, 6)


def kernel(x, Wq, K_ext, V_ext, Wo):
    d_model = x.shape[-1]

    def body(x_ref, wq_ref, k_ref, v_ref, wo_ref, out_ref,
             nb_ref, q32_ref, k0_ref, part_ref, parts0_ref,
             num32_ref, den32_ref,
             nbr_r_send, nbr_r_recv, nbr_l_send, nbr_l_recv,
             q32_send, q32_recv, k0_send, k0_recv,
             part_send, part_recv):
        my = lax.axis_index("i")
        left = lax.rem(my - 1 + N_DEV, N_DEV)
        right = lax.rem(my + 1, N_DEV)
        is_zero = my == 0
        is_mid = (my >= 2) & (my <= 6)

        barrier_sem = pltpu.get_barrier_semaphore()
        for nbr in (left, right):
            pl.semaphore_signal(
                barrier_sem, inc=1,
                device_id=(nbr,), device_id_type=pl.DeviceIdType.MESH,
            )

        @pl.when(is_zero)
        def _():
            for d in MIDS:
                pl.semaphore_signal(
                    barrier_sem, inc=1,
                    device_id=(d,), device_id_type=pl.DeviceIdType.MESH,
                )

        @pl.when(is_mid)
        def _():
            pl.semaphore_signal(
                barrier_sem, inc=1,
                device_id=(0,), device_id_type=pl.DeviceIdType.MESH,
            )

        pl.semaphore_wait(barrier_sem, 2)

        @pl.when(is_zero)
        def _():
            pl.semaphore_wait(barrier_sem, len(MIDS))

        @pl.when(is_mid)
        def _():
            pl.semaphore_wait(barrier_sem, 1)

        for j in range(HQ):
            nb_ref[pl.ds(j * B, B), pl.ds(S_PER, S_PER), :] = (
                k_ref[:, :, j, :].astype(jnp.bfloat16))
            nb_ref[pl.ds(G + j * B, B), pl.ds(S_PER, S_PER), :] = (
                v_ref[:, :, j, :].astype(jnp.bfloat16))

        send_r = pltpu.make_async_remote_copy(
            src_ref=nb_ref.at[:, pl.ds(S_PER, S_PER), :],
            dst_ref=nb_ref.at[:, pl.ds(0, S_PER), :],
            send_sem=nbr_r_send.at[0], recv_sem=nbr_r_recv.at[0],
            device_id=(right,), device_id_type=pl.DeviceIdType.MESH,
        )
        send_l = pltpu.make_async_remote_copy(
            src_ref=nb_ref.at[:, pl.ds(S_PER, S_PER), :],
            dst_ref=nb_ref.at[:, pl.ds(2 * S_PER, S_PER), :],
            send_sem=nbr_l_send.at[0], recv_sem=nbr_l_recv.at[0],
            device_id=(left,), device_id_type=pl.DeviceIdType.MESH,
        )
        send_r.start()
        send_l.start()

        xb = x_ref[:].astype(jnp.bfloat16)
        q = lax.dot_general(
            xb, wq_ref[:].astype(jnp.bfloat16), (((2,), (0,)), ((), ())),
            preferred_element_type=jnp.float32,
        )
        qall = jnp.concatenate(
            [q[:, :, h * DH:(h + 1) * DH] for h in range(HQ)], axis=0
        ).astype(jnp.bfloat16)

        q32_rdmas = []
        k0_rdmas = []
        for i, d in enumerate(MIDS):
            q32_rdmas.append(pltpu.make_async_remote_copy(
                src_ref=q32_ref, dst_ref=q32_ref,
                send_sem=q32_send.at[i], recv_sem=q32_recv.at[0],
                device_id=(d,), device_id_type=pl.DeviceIdType.MESH,
            ))
            k0_rdmas.append(pltpu.make_async_remote_copy(
                src_ref=nb_ref.at[:, pl.ds(S_PER, GLOBAL_PREFIX), :],
                dst_ref=k0_ref,
                send_sem=k0_send.at[i], recv_sem=k0_recv.at[0],
                device_id=(d,), device_id_type=pl.DeviceIdType.MESH,
            ))

        @pl.when(is_zero)
        def _():
            q32_ref[:] = qall[:, :GLOBAL_PREFIX, :]
            for r in q32_rdmas:
                r.start()
            for r in k0_rdmas:
                r.start()

        @pl.when(is_mid)
        def _():
            q32_rdmas[0].wait_recv()

        kb_own = nb_ref[0:G, pl.ds(S_PER, S_PER), :]
        vb_own = nb_ref[G:2 * G, pl.ds(S_PER, S_PER), :]
        s32 = lax.dot_general(
            q32_ref[:], kb_own, (((2,), (2,)), ((0,), (0,))),
            preferred_element_type=jnp.float32,
        ) * 0.125
        e32 = jnp.exp(s32)
        p_num = lax.dot_general(
            e32.astype(jnp.bfloat16), vb_own, (((2,), (1,)), ((0,), (0,))),
            preferred_element_type=jnp.float32,
        )
        p_den = jnp.sum(e32, axis=-1, keepdims=True)
        part_ref[0] = p_num
        part_ref[1] = jnp.broadcast_to(p_den, (G, GLOBAL_PREFIX, DH))

        part_rdma = pltpu.make_async_remote_copy(
            src_ref=part_ref, dst_ref=parts0_ref.at[my - 2],
            send_sem=part_send.at[0], recv_sem=part_recv.at[my - 2],
            device_id=(0,), device_id_type=pl.DeviceIdType.MESH,
        )

        @pl.when(is_mid)
        def _():
            part_rdma.start()

        send_r.wait_recv()
        send_l.wait_recv()

        qi_glob = (lax.broadcasted_iota(jnp.int32, (S_PER, S_NEAR), 0)
                   + my * S_PER)
        fake_ki = (lax.broadcasted_iota(jnp.int32, (S_PER, S_NEAR), 1)
                   + (my - 1) * S_PER)
        real_ki = lax.rem(fake_ki + S_GLOB, S_GLOB)
        mask = ((jnp.abs(qi_glob - real_ki) <= LOCAL_WINDOW)
                | (real_ki < GLOBAL_PREFIX) | (qi_glob < GLOBAL_PREFIX))
        maskf = mask.astype(jnp.float32)[None, :, :]

        kb = nb_ref[0:G]
        vb = nb_ref[G:2 * G]
        s = lax.dot_general(
            qall, kb, (((2,), (2,)), ((0,), (0,))),
            preferred_element_type=jnp.float32,
        ) * 0.125
        e = jnp.exp(s) * maskf
        num = lax.dot_general(
            e.astype(jnp.bfloat16), vb, (((2,), (1,)), ((0,), (0,))),
            preferred_element_type=jnp.float32,
        )
        den = jnp.sum(e, axis=-1, keepdims=True)

        @pl.when(is_mid)
        def _():
            k0_rdmas[0].wait_recv()

        sb = lax.dot_general(
            qall, k0_ref[0:G], (((2,), (2,)), ((0,), (0,))),
            preferred_element_type=jnp.float32,
        ) * 0.125
        eb = jnp.where(is_mid, jnp.exp(sb), 0.0)
        num = num + lax.dot_general(
            eb.astype(jnp.bfloat16), k0_ref[G:2 * G],
            (((2,), (1,)), ((0,), (0,))),
            preferred_element_type=jnp.float32,
        )
        den = den + jnp.sum(eb, axis=-1, keepdims=True)

        @pl.when(is_zero)
        def _():
            for i in range(len(MIDS)):
                part_recv_desc = pltpu.make_async_remote_copy(
                    src_ref=part_ref, dst_ref=parts0_ref.at[i],
                    send_sem=part_send.at[0], recv_sem=part_recv.at[i],
                    device_id=(0,), device_id_type=pl.DeviceIdType.MESH,
                )
                part_recv_desc.wait_recv()
            n32 = parts0_ref[0, 0]
            d32 = parts0_ref[0, 1, :, :, 0:1]
            for i in range(1, len(MIDS)):
                n32 = n32 + parts0_ref[i, 0]
                d32 = d32 + parts0_ref[i, 1, :, :, 0:1]
            num32_ref[:] = n32
            den32_ref[:] = d32

        @pl.when(jnp.logical_not(is_zero))
        def _():
            num32_ref[:] = jnp.zeros((G, GLOBAL_PREFIX, DH), jnp.float32)
            den32_ref[:] = jnp.zeros((G, GLOBAL_PREFIX, 1), jnp.float32)

        n = jnp.concatenate(
            [num[:, :GLOBAL_PREFIX, :] + num32_ref[:],
             num[:, GLOBAL_PREFIX:, :]], axis=1)
        d = jnp.concatenate(
            [den[:, :GLOBAL_PREFIX, :] + den32_ref[:],
             den[:, GLOBAL_PREFIX:, :]], axis=1)

        ctx = (n / d).astype(jnp.bfloat16)
        acc = jnp.zeros((B, S_PER, d_model), dtype=jnp.float32)
        for h in range(HQ):
            woh = wo_ref[h * DH:(h + 1) * DH, :].astype(jnp.bfloat16)
            acc = acc + lax.dot_general(
                ctx[h * B:(h + 1) * B], woh, (((2,), (0,)), ((), ())),
                preferred_element_type=jnp.float32,
            )
        out_ref[:] = acc

        send_r.wait_send()
        send_l.wait_send()

        @pl.when(is_zero)
        def _():
            for r in q32_rdmas:
                r.wait_send()
            for r in k0_rdmas:
                r.wait_send()

        @pl.when(is_mid)
        def _():
            part_rdma.wait_send()

    out_shape = jax.ShapeDtypeStruct((B, S_PER, d_model), jnp.float32)
    return pl.pallas_call(
        body,
        out_shape=out_shape,
        in_specs=[pl.BlockSpec(memory_space=pltpu.VMEM)] * 5,
        out_specs=pl.BlockSpec(memory_space=pltpu.VMEM),
        scratch_shapes=[
            pltpu.VMEM((2 * G, S_NEAR, DH), jnp.bfloat16),
            pltpu.VMEM((G, GLOBAL_PREFIX, DH), jnp.bfloat16),
            pltpu.VMEM((2 * G, GLOBAL_PREFIX, DH), jnp.bfloat16),
            pltpu.VMEM((2, G, GLOBAL_PREFIX, DH), jnp.float32),
            pltpu.VMEM((len(MIDS), 2, G, GLOBAL_PREFIX, DH),
                       jnp.float32),
            pltpu.VMEM((G, GLOBAL_PREFIX, DH), jnp.float32),
            pltpu.VMEM((G, GLOBAL_PREFIX, 1), jnp.float32),
            pltpu.SemaphoreType.DMA((1,)),
            pltpu.SemaphoreType.DMA((1,)),
            pltpu.SemaphoreType.DMA((1,)),
            pltpu.SemaphoreType.DMA((1,)),
            pltpu.SemaphoreType.DMA((len(MIDS),)),
            pltpu.SemaphoreType.DMA((1,)),
            pltpu.SemaphoreType.DMA((len(MIDS),)),
            pltpu.SemaphoreType.DMA((1,)),
            pltpu.SemaphoreType.DMA((1,)),
            pltpu.SemaphoreType.DMA((len(MIDS),)),
        ],
        compiler_params=pltpu.CompilerParams(collective_id=0),
    )(x, Wq, K_ext, V_ext, Wo)


# device time: 24891 ns/iter; 1.6276x vs baseline; 1.1542x over previous
import jax
import jax.numpy as jnp
from jax import lax
from jax.experimental import pallas as pl
from jax.experimental.pallas import tpu as pltpu

N_DEV = 8
B = 2
S_PER = 128
S_GLOB = N_DEV * S_PER
HQ = 4
DH = 64
G = HQ * B
S_NEAR = 3 * S_PER
LOCAL_WINDOW = 128
GLOBAL_PREFIX = 32
MIDS = (2, 3, 4, 5, 6)


def kernel(x, Wq, K_ext, V_ext, Wo):
    d_model = x.shape[-1]

    def body(x_ref, wq_ref, k_ref, v_ref, wo_ref, out_ref,
             nb_ref, q32_ref, k0_ref, part_ref, parts0_ref,
             num32_ref, den32_ref,
             nbr_r_send, nbr_r_recv, nbr_l_send, nbr_l_recv,
             q32_send, q32_recv, k0_send, k0_recv,
             part_send, part_recv):
        my = lax.axis_index("i")
        left = lax.rem(my - 1 + N_DEV, N_DEV)
        right = lax.rem(my + 1, N_DEV)
        is_zero = my == 0
        is_mid = (my >= 2) & (my <= 6)

        barrier_sem = pltpu.get_barrier_semaphore()
        for nbr in (left, right):
            pl.semaphore_signal(
                barrier_sem, inc=1,
                device_id=(nbr,), device_id_type=pl.DeviceIdType.MESH,
            )

        @pl.when(is_zero)
        def _():
            for d in MIDS:
                pl.semaphore_signal(
                    barrier_sem, inc=1,
                    device_id=(d,), device_id_type=pl.DeviceIdType.MESH,
                )

        @pl.when(is_mid)
        def _():
            pl.semaphore_signal(
                barrier_sem, inc=1,
                device_id=(0,), device_id_type=pl.DeviceIdType.MESH,
            )

        pl.semaphore_wait(barrier_sem, 2)

        @pl.when(is_zero)
        def _():
            pl.semaphore_wait(barrier_sem, len(MIDS))

        @pl.when(is_mid)
        def _():
            pl.semaphore_wait(barrier_sem, 1)

        for j in range(HQ):
            nb_ref[pl.ds(j * B, B), pl.ds(S_PER, S_PER), :] = (
                k_ref[:, :, j, :].astype(jnp.bfloat16))
            nb_ref[pl.ds(G + j * B, B), pl.ds(S_PER, S_PER), :] = (
                v_ref[:, :, j, :].astype(jnp.bfloat16))

        send_r = pltpu.make_async_remote_copy(
            src_ref=nb_ref.at[:, pl.ds(S_PER, S_PER), :],
            dst_ref=nb_ref.at[:, pl.ds(0, S_PER), :],
            send_sem=nbr_r_send.at[0], recv_sem=nbr_r_recv.at[0],
            device_id=(right,), device_id_type=pl.DeviceIdType.MESH,
        )
        send_l = pltpu.make_async_remote_copy(
            src_ref=nb_ref.at[:, pl.ds(S_PER, S_PER), :],
            dst_ref=nb_ref.at[:, pl.ds(2 * S_PER, S_PER), :],
            send_sem=nbr_l_send.at[0], recv_sem=nbr_l_recv.at[0],
            device_id=(left,), device_id_type=pl.DeviceIdType.MESH,
        )
        send_r.start()
        send_l.start()

        xb = x_ref[:].astype(jnp.bfloat16)
        wqb = wq_ref[:].astype(jnp.bfloat16)

        q32_rdmas = []
        k0_rdmas = []
        for i, d in enumerate(MIDS):
            q32_rdmas.append(pltpu.make_async_remote_copy(
                src_ref=q32_ref, dst_ref=q32_ref,
                send_sem=q32_send.at[i], recv_sem=q32_recv.at[0],
                device_id=(d,), device_id_type=pl.DeviceIdType.MESH,
            ))
            k0_rdmas.append(pltpu.make_async_remote_copy(
                src_ref=nb_ref.at[:, pl.ds(S_PER, GLOBAL_PREFIX), :],
                dst_ref=k0_ref,
                send_sem=k0_send.at[i], recv_sem=k0_recv.at[0],
                device_id=(d,), device_id_type=pl.DeviceIdType.MESH,
            ))

        @pl.when(is_zero)
        def _():
            q32 = lax.dot_general(
                x_ref[:, :GLOBAL_PREFIX, :].astype(jnp.bfloat16), wqb,
                (((2,), (0,)), ((), ())),
                preferred_element_type=jnp.float32,
            )
            q32_ref[:] = jnp.concatenate(
                [q32[:, :, h * DH:(h + 1) * DH] for h in range(HQ)], axis=0
            ).astype(jnp.bfloat16)
            for r in q32_rdmas:
                r.start()
            for r in k0_rdmas:
                r.start()

        q = lax.dot_general(
            xb, wqb, (((2,), (0,)), ((), ())),
            preferred_element_type=jnp.float32,
        )
        qall = jnp.concatenate(
            [q[:, :, h * DH:(h + 1) * DH] for h in range(HQ)], axis=0
        ).astype(jnp.bfloat16)

        part_rdma = pltpu.make_async_remote_copy(
            src_ref=part_ref, dst_ref=parts0_ref.at[my - 2],
            send_sem=part_send.at[0], recv_sem=part_recv.at[my - 2],
            device_id=(0,), device_id_type=pl.DeviceIdType.MESH,
        )

        @pl.when(is_mid)
        def _():
            q32_rdmas[0].wait_recv()
            kb_own = nb_ref[0:G, pl.ds(S_PER, S_PER), :]
            vb_own = nb_ref[G:2 * G, pl.ds(S_PER, S_PER), :]
            s32 = lax.dot_general(
                q32_ref[:], kb_own, (((2,), (2,)), ((0,), (0,))),
                preferred_element_type=jnp.float32,
            ) * 0.125
            e32 = jnp.exp(s32)
            p_num = lax.dot_general(
                e32.astype(jnp.bfloat16), vb_own,
                (((2,), (1,)), ((0,), (0,))),
                preferred_element_type=jnp.float32,
            )
            p_den = jnp.sum(e32, axis=-1, keepdims=True)
            part_ref[0] = p_num.astype(jnp.bfloat16)
            part_ref[1] = jnp.broadcast_to(
                p_den, (G, GLOBAL_PREFIX, DH)).astype(jnp.bfloat16)
            part_rdma.start()

        send_r.wait_recv()
        send_l.wait_recv()

        qi_glob = (lax.broadcasted_iota(jnp.int32, (S_PER, S_NEAR), 0)
                   + my * S_PER)
        fake_ki = (lax.broadcasted_iota(jnp.int32, (S_PER, S_NEAR), 1)
                   + (my - 1) * S_PER)
        real_ki = lax.rem(fake_ki + S_GLOB, S_GLOB)
        mask = ((jnp.abs(qi_glob - real_ki) <= LOCAL_WINDOW)
                | (real_ki < GLOBAL_PREFIX) | (qi_glob < GLOBAL_PREFIX))
        maskf = mask.astype(jnp.float32)[None, :, :]

        kb = nb_ref[0:G]
        vb = nb_ref[G:2 * G]
        s = lax.dot_general(
            qall, kb, (((2,), (2,)), ((0,), (0,))),
            preferred_element_type=jnp.float32,
        ) * 0.125
        e = jnp.exp(s) * maskf
        num = lax.dot_general(
            e.astype(jnp.bfloat16), vb, (((2,), (1,)), ((0,), (0,))),
            preferred_element_type=jnp.float32,
        )
        den = jnp.sum(e, axis=-1, keepdims=True)

        @pl.when(is_mid)
        def _():
            k0_rdmas[0].wait_recv()

        sb = lax.dot_general(
            qall, k0_ref[0:G], (((2,), (2,)), ((0,), (0,))),
            preferred_element_type=jnp.float32,
        ) * 0.125
        eb = jnp.where(is_mid, jnp.exp(sb), 0.0)
        num = num + lax.dot_general(
            eb.astype(jnp.bfloat16), k0_ref[G:2 * G],
            (((2,), (1,)), ((0,), (0,))),
            preferred_element_type=jnp.float32,
        )
        den = den + jnp.sum(eb, axis=-1, keepdims=True)

        @pl.when(is_zero)
        def _():
            for i in range(len(MIDS)):
                part_recv_desc = pltpu.make_async_remote_copy(
                    src_ref=part_ref, dst_ref=parts0_ref.at[i],
                    send_sem=part_send.at[0], recv_sem=part_recv.at[i],
                    device_id=(0,), device_id_type=pl.DeviceIdType.MESH,
                )
                part_recv_desc.wait_recv()
            n32 = parts0_ref[0, 0].astype(jnp.float32)
            d32 = parts0_ref[0, 1, :, :, 0:1].astype(jnp.float32)
            for i in range(1, len(MIDS)):
                n32 = n32 + parts0_ref[i, 0].astype(jnp.float32)
                d32 = d32 + parts0_ref[i, 1, :, :, 0:1].astype(jnp.float32)
            num32_ref[:] = n32
            den32_ref[:] = d32

        @pl.when(jnp.logical_not(is_zero))
        def _():
            num32_ref[:] = jnp.zeros((G, GLOBAL_PREFIX, DH), jnp.float32)
            den32_ref[:] = jnp.zeros((G, GLOBAL_PREFIX, 1), jnp.float32)

        n = jnp.concatenate(
            [num[:, :GLOBAL_PREFIX, :] + num32_ref[:],
             num[:, GLOBAL_PREFIX:, :]], axis=1)
        d = jnp.concatenate(
            [den[:, :GLOBAL_PREFIX, :] + den32_ref[:],
             den[:, GLOBAL_PREFIX:, :]], axis=1)

        ctx = (n / d).astype(jnp.bfloat16)
        acc = jnp.zeros((B, S_PER, d_model), dtype=jnp.float32)
        for h in range(HQ):
            woh = wo_ref[h * DH:(h + 1) * DH, :].astype(jnp.bfloat16)
            acc = acc + lax.dot_general(
                ctx[h * B:(h + 1) * B], woh, (((2,), (0,)), ((), ())),
                preferred_element_type=jnp.float32,
            )
        out_ref[:] = acc

        send_r.wait_send()
        send_l.wait_send()

        @pl.when(is_zero)
        def _():
            for r in q32_rdmas:
                r.wait_send()
            for r in k0_rdmas:
                r.wait_send()

        @pl.when(is_mid)
        def _():
            part_rdma.wait_send()

    out_shape = jax.ShapeDtypeStruct((B, S_PER, d_model), jnp.float32)
    return pl.pallas_call(
        body,
        out_shape=out_shape,
        in_specs=[pl.BlockSpec(memory_space=pltpu.VMEM)] * 5,
        out_specs=pl.BlockSpec(memory_space=pltpu.VMEM),
        scratch_shapes=[
            pltpu.VMEM((2 * G, S_NEAR, DH), jnp.bfloat16),
            pltpu.VMEM((G, GLOBAL_PREFIX, DH), jnp.bfloat16),
            pltpu.VMEM((2 * G, GLOBAL_PREFIX, DH), jnp.bfloat16),
            pltpu.VMEM((2, G, GLOBAL_PREFIX, DH), jnp.bfloat16),
            pltpu.VMEM((len(MIDS), 2, G, GLOBAL_PREFIX, DH),
                       jnp.bfloat16),
            pltpu.VMEM((G, GLOBAL_PREFIX, DH), jnp.float32),
            pltpu.VMEM((G, GLOBAL_PREFIX, 1), jnp.float32),
            pltpu.SemaphoreType.DMA((1,)),
            pltpu.SemaphoreType.DMA((1,)),
            pltpu.SemaphoreType.DMA((1,)),
            pltpu.SemaphoreType.DMA((1,)),
            pltpu.SemaphoreType.DMA((len(MIDS),)),
            pltpu.SemaphoreType.DMA((1,)),
            pltpu.SemaphoreType.DMA((len(MIDS),)),
            pltpu.SemaphoreType.DMA((1,)),
            pltpu.SemaphoreType.DMA((1,)),
            pltpu.SemaphoreType.DMA((len(MIDS),)),
        ],
        compiler_params=pltpu.CompilerParams(collective_id=0),
    )(x, Wq, K_ext, V_ext, Wo)
